# baseline (speedup 1.0000x reference)
import functools
import math

import jax
import jax.numpy as jnp
from jax import lax
from jax.experimental import pallas as pl
from jax.experimental.pallas import tpu as pltpu
from jax.experimental.pallas import tpu_sc as plsc

HID = 128
NRBF = 50
RCUT = 5.0
LN2 = 0.6931471805599453

NCORE = 2
NSUB = 16
NTILE = NCORE * NSUB
CHUNK = 96

BE = 2048


def _ssp(x):
    m = jnp.maximum(x, 0.0)
    return m + jnp.log(jnp.exp(x - m) + jnp.exp(-m)) - LN2


def _filters_body(nl, ev_ref, w1_ref, b1_ref, w2_ref, b2_ref, *out_refs):
    ev = ev_ref[...]
    d = jnp.sqrt(jnp.sum(ev * ev, axis=1, keepdims=True))
    delta = RCUT / (NRBF - 1)
    offs = lax.broadcasted_iota(jnp.int32, (1, NRBF), 1).astype(jnp.float32) * delta
    coeff = -0.5 / (delta * delta)
    rbf = jnp.exp(coeff * (d - offs) ** 2)
    cutoff = 0.5 * (jnp.cos(d * (jnp.pi / RCUT)) + 1.0)
    cutoff = cutoff * (d < RCUT).astype(jnp.float32)
    for i in range(nl):
        t = jnp.dot(rbf, w1_ref[i], preferred_element_type=jnp.float32)
        t = _ssp(t + b1_ref[i])
        w = jnp.dot(t, w2_ref[i], preferred_element_type=jnp.float32)
        w = (w + b2_ref[i]) * cutoff
        out_refs[i][...] = w


def _compute_filters(evp, w1, b1, w2, b2):
    nl = w1.shape[0]
    epad = evp.shape[0]
    grid = (epad // BE,)
    out_shape = tuple(
        jax.ShapeDtypeStruct((epad, HID), jnp.float32) for _ in range(nl))
    return pl.pallas_call(
        functools.partial(_filters_body, nl),
        grid=grid,
        in_specs=[
            pl.BlockSpec((BE, 3), lambda i: (i, 0)),
            pl.BlockSpec(w1.shape, lambda i: (0, 0, 0)),
            pl.BlockSpec(b1.shape, lambda i: (0, 0)),
            pl.BlockSpec(w2.shape, lambda i: (0, 0, 0)),
            pl.BlockSpec(b2.shape, lambda i: (0, 0)),
        ],
        out_specs=tuple(
            pl.BlockSpec((BE, HID), lambda i: (i, 0)) for _ in range(nl)),
        out_shape=out_shape,
    )(evp, w1, b1, w2, b2)


def _embed_body(z_ref, emb_ref, l1_ref, h_ref, x_ref):
    zb = z_ref[...]
    nv = emb_ref.shape[0]
    oh = (zb == lax.broadcasted_iota(jnp.int32, (1, nv), 1)).astype(jnp.float32)
    h = jnp.dot(oh, emb_ref[...], preferred_element_type=jnp.float32)
    h_ref[...] = h
    x_ref[...] = jnp.dot(h, l1_ref[...], preferred_element_type=jnp.float32)


def _embed(z2, emb, l1, bn):
    n = z2.shape[0]
    return pl.pallas_call(
        _embed_body,
        grid=(n // bn,),
        in_specs=[
            pl.BlockSpec((bn, 1), lambda i: (i, 0)),
            pl.BlockSpec(emb.shape, lambda i: (0, 0)),
            pl.BlockSpec(l1.shape, lambda i: (0, 0)),
        ],
        out_specs=(
            pl.BlockSpec((bn, HID), lambda i: (i, 0)),
            pl.BlockSpec((bn, HID), lambda i: (i, 0)),
        ),
        out_shape=(
            jax.ShapeDtypeStruct((n, HID), jnp.float32),
            jax.ShapeDtypeStruct((n, HID), jnp.float32),
        ),
    )(z2, emb, l1)


def _make_msg_sc(n, epad, cpt):
    rps = n // NSUB
    mesh = plsc.VectorSubcoreMesh(core_axis_name="c", subcore_axis_name="s",
                                  num_cores=NCORE, num_subcores=NSUB)

    @functools.partial(
        pl.kernel,
        out_type=jax.ShapeDtypeStruct((NCORE, n, HID), jnp.float32),
        mesh=mesh,
        scratch_types=[
            pltpu.VMEM_SHARED((n, HID), jnp.float32),
            pltpu.VMEM((2, CHUNK), jnp.int32),
            pltpu.VMEM((2, CHUNK), jnp.int32),
            pltpu.VMEM((CHUNK, HID), jnp.float32),
            pltpu.VMEM((CHUNK, HID), jnp.float32),
            pltpu.VMEM((CHUNK, HID), jnp.float32),
            pltpu.VMEM((CHUNK, HID), jnp.float32),
            pltpu.SemaphoreType.DMA,
            pltpu.SemaphoreType.DMA,
            pltpu.SemaphoreType.DMA,
            pltpu.SemaphoreType.DMA,
        ],
    )
    def msg_sc(x_hbm, w_hbm, src_hbm, dst_hbm, zero_hbm, out_hbm,
               agg_sp, srcv, dstv, xv0, xv1, wv0, wv1, sg0, sg1, ss0, ss1):
        c = lax.axis_index("c")
        s = lax.axis_index("s")
        wid = c * NSUB + s
        xv = (xv0, xv1)
        wv = (wv0, wv1)
        sg = (sg0, sg1)
        ss = (ss0, ss1)

        row0 = s * rps
        rsl = pl.ds(row0, rps)
        pltpu.sync_copy(zero_hbm.at[rsl], agg_sp.at[rsl])
        plsc.subcore_barrier()

        base0 = wid * cpt * CHUNK

        def chunk_step(k, b, do_swait, do_issue):
            eb = base0 + k * CHUNK
            pltpu.make_async_copy(x_hbm.at[srcv.at[b]], xv[b], sg[b]).wait()
            if do_swait:
                pltpu.make_async_copy(wv[b], agg_sp.at[dstv.at[b]],
                                      ss[b]).wait()
            pltpu.sync_copy(w_hbm.at[pl.ds(eb, CHUNK)], wv[b])

            def mbody(e, cc):
                for j in range(HID // 16):
                    sl = pl.ds(j * 16, 16)
                    wv[b][e, sl] = wv[b][e, sl] * xv[b][e, sl]
                return cc

            lax.fori_loop(0, CHUNK, mbody, 0)
            pltpu.sync_copy(dst_hbm.at[pl.ds(eb, CHUNK)], dstv.at[b])
            pltpu.async_copy(wv[b], agg_sp.at[dstv.at[b]], ss[b], add=True)
            if do_issue:
                eb2 = base0 + (k + 2) * CHUNK
                pltpu.sync_copy(src_hbm.at[pl.ds(eb2, CHUNK)], srcv.at[b])
                pltpu.async_copy(x_hbm.at[srcv.at[b]], xv[b], sg[b])

        for b in range(2):
            pltpu.sync_copy(src_hbm.at[pl.ds(base0 + b * CHUNK, CHUNK)],
                            srcv.at[b])
            pltpu.async_copy(x_hbm.at[srcv.at[b]], xv[b], sg[b])
        for k in range(min(2, cpt)):
            chunk_step(k, k % 2, False, k < cpt - 2)
        npairs = max(0, (cpt - 4) // 2)

        def pbody(g, carry):
            k0 = 2 + 2 * g
            chunk_step(k0, 0, True, True)
            chunk_step(k0 + 1, 1, True, True)
            return carry

        lax.fori_loop(0, npairs, pbody, 0)
        for k in range(2 + 2 * npairs, cpt):
            chunk_step(k, k % 2, True, k < cpt - 2)
        for k in range(max(0, cpt - 2), cpt):
            b = k % 2
            pltpu.make_async_copy(wv[b], agg_sp.at[dstv.at[b]], ss[b]).wait()
        plsc.subcore_barrier()

        pltpu.sync_copy(agg_sp.at[rsl], out_hbm.at[c, rsl])

    return msg_sc


def _update_body(p_ref, h_ref, l2_ref, l2b_ref, ow_ref, ob_ref, l1n_ref,
                 hout_ref, xout_ref):
    agg = p_ref[0] + p_ref[1]
    y = jnp.dot(agg, l2_ref[...], preferred_element_type=jnp.float32)
    y = _ssp(y + l2b_ref[...])
    y = jnp.dot(y, ow_ref[...], preferred_element_type=jnp.float32) + ob_ref[...]
    h = h_ref[...] + y
    hout_ref[...] = h
    xout_ref[...] = jnp.dot(h, l1n_ref[...], preferred_element_type=jnp.float32)


def _update(part, h, l2, l2b, ow, ob, l1n, bn):
    n = h.shape[0]
    return pl.pallas_call(
        _update_body,
        grid=(n // bn,),
        in_specs=[
            pl.BlockSpec((NCORE, bn, HID), lambda i: (0, i, 0)),
            pl.BlockSpec((bn, HID), lambda i: (i, 0)),
            pl.BlockSpec((HID, HID), lambda i: (0, 0)),
            pl.BlockSpec((1, HID), lambda i: (0, 0)),
            pl.BlockSpec((HID, HID), lambda i: (0, 0)),
            pl.BlockSpec((1, HID), lambda i: (0, 0)),
            pl.BlockSpec((HID, HID), lambda i: (0, 0)),
        ],
        out_specs=(
            pl.BlockSpec((bn, HID), lambda i: (i, 0)),
            pl.BlockSpec((bn, HID), lambda i: (i, 0)),
        ),
        out_shape=(
            jax.ShapeDtypeStruct((n, HID), jnp.float32),
            jax.ShapeDtypeStruct((n, HID), jnp.float32),
        ),
    )(part, h, l2, l2b, ow, ob, l1n)


def kernel(z, edge_index, edge_vec, emb, mlp_W1, mlp_b1, mlp_W2, mlp_b2,
           lin1_W, lin2_W, lin2_b, out_W, out_b):
    n = z.shape[0]
    e = edge_index.shape[1]
    nl = mlp_W1.shape[0]

    z = z.astype(jnp.int32)
    edge_index = edge_index.astype(jnp.int32)
    edge_vec = edge_vec.astype(jnp.float32)

    egrain = (NTILE * CHUNK) * BE // math.gcd(NTILE * CHUNK, BE)
    epad = -(-e // egrain) * egrain
    cpt = epad // (NTILE * CHUNK)
    pad = epad - e
    src = jnp.pad(edge_index[0], (0, pad))
    dst = jnp.pad(edge_index[1], (0, pad))
    evp = jnp.pad(edge_vec, ((0, pad), (0, 0)), constant_values=10.0 * RCUT)

    w_layers = _compute_filters(evp, mlp_W1, mlp_b1, mlp_W2, mlp_b2)

    npad = -(-n // (NSUB * 8)) * (NSUB * 8)
    bn = npad // 4 if (npad // 4) % 8 == 0 else npad // NSUB
    zp = jnp.pad(z, (0, npad - n), constant_values=emb.shape[0] + 7)
    h, x = _embed(zp.reshape(npad, 1), emb, lin1_W[0], bn)

    msg_sc = _make_msg_sc(npad, epad, cpt)
    zero_tab = jnp.zeros((npad, HID), jnp.float32)
    for i in range(nl):
        part = msg_sc(x, w_layers[i], src, dst, zero_tab)
        l1n = lin1_W[(i + 1) % nl]
        h, x = _update(part, h, lin2_W[i], lin2_b[i].reshape(1, HID),
                       out_W[i], out_b[i].reshape(1, HID), l1n, bn)
    return h[:n]


# speedup vs baseline: 1.0559x; 1.0559x over previous
import functools
import math

import jax
import jax.numpy as jnp
from jax import lax
from jax.experimental import pallas as pl
from jax.experimental.pallas import tpu as pltpu
from jax.experimental.pallas import tpu_sc as plsc

HID = 128
NRBF = 50
RCUT = 5.0
LN2 = 0.6931471805599453

NCORE = 2
NSUB = 16
NTILE = NCORE * NSUB
CHUNK = 96

BE = 2048


def _ssp(x):
    m = jnp.maximum(x, 0.0)
    return m + jnp.log(jnp.exp(x - m) + jnp.exp(-m)) - LN2


def _filters_body(nl, ev_ref, w1_ref, b1_ref, w2_ref, b2_ref, *out_refs):
    ev = ev_ref[...]
    d = jnp.sqrt(jnp.sum(ev * ev, axis=1, keepdims=True))
    delta = RCUT / (NRBF - 1)
    offs = lax.broadcasted_iota(jnp.int32, (1, NRBF), 1).astype(jnp.float32) * delta
    coeff = -0.5 / (delta * delta)
    rbf = jnp.exp(coeff * (d - offs) ** 2)
    cutoff = 0.5 * (jnp.cos(d * (jnp.pi / RCUT)) + 1.0)
    cutoff = cutoff * (d < RCUT).astype(jnp.float32)
    for i in range(nl):
        t = jnp.dot(rbf, w1_ref[i], preferred_element_type=jnp.float32)
        t = _ssp(t + b1_ref[i])
        w = jnp.dot(t, w2_ref[i], preferred_element_type=jnp.float32)
        w = (w + b2_ref[i]) * cutoff
        out_refs[i][...] = w


def _compute_filters(evp, w1, b1, w2, b2):
    nl = w1.shape[0]
    epad = evp.shape[0]
    grid = (epad // BE,)
    out_shape = tuple(
        jax.ShapeDtypeStruct((epad, HID), jnp.float32) for _ in range(nl))
    return pl.pallas_call(
        functools.partial(_filters_body, nl),
        grid=grid,
        in_specs=[
            pl.BlockSpec((BE, 3), lambda i: (i, 0)),
            pl.BlockSpec(w1.shape, lambda i: (0, 0, 0)),
            pl.BlockSpec(b1.shape, lambda i: (0, 0)),
            pl.BlockSpec(w2.shape, lambda i: (0, 0, 0)),
            pl.BlockSpec(b2.shape, lambda i: (0, 0)),
        ],
        out_specs=tuple(
            pl.BlockSpec((BE, HID), lambda i: (i, 0)) for _ in range(nl)),
        out_shape=out_shape,
    )(evp, w1, b1, w2, b2)


def _embed_body(z_ref, emb_ref, l1_ref, h_ref, x_ref):
    zb = z_ref[...]
    nv = emb_ref.shape[0]
    oh = (zb == lax.broadcasted_iota(jnp.int32, (1, nv), 1)).astype(jnp.float32)
    h = jnp.dot(oh, emb_ref[...], preferred_element_type=jnp.float32)
    h_ref[...] = h
    x_ref[...] = jnp.dot(h, l1_ref[...], preferred_element_type=jnp.float32)


def _embed(z2, emb, l1, bn):
    n = z2.shape[0]
    return pl.pallas_call(
        _embed_body,
        grid=(n // bn,),
        in_specs=[
            pl.BlockSpec((bn, 1), lambda i: (i, 0)),
            pl.BlockSpec(emb.shape, lambda i: (0, 0)),
            pl.BlockSpec(l1.shape, lambda i: (0, 0)),
        ],
        out_specs=(
            pl.BlockSpec((bn, HID), lambda i: (i, 0)),
            pl.BlockSpec((bn, HID), lambda i: (i, 0)),
        ),
        out_shape=(
            jax.ShapeDtypeStruct((n, HID), jnp.float32),
            jax.ShapeDtypeStruct((n, HID), jnp.float32),
        ),
    )(z2, emb, l1)


def _make_msg_sc(n, epad, cpt):
    rps = n // NSUB
    mesh = plsc.VectorSubcoreMesh(core_axis_name="c", subcore_axis_name="s",
                                  num_cores=NCORE, num_subcores=NSUB)

    @functools.partial(
        pl.kernel,
        out_type=jax.ShapeDtypeStruct((NCORE, n, HID), jnp.float32),
        mesh=mesh,
        scratch_types=[
            pltpu.VMEM_SHARED((n, HID), jnp.float32),
            pltpu.VMEM((2, 2, CHUNK), jnp.int32),
            pltpu.VMEM((2, 2, CHUNK), jnp.int32),
            pltpu.VMEM((CHUNK, HID), jnp.float32),
            pltpu.VMEM((CHUNK, HID), jnp.float32),
            pltpu.VMEM((CHUNK, HID), jnp.float32),
            pltpu.VMEM((CHUNK, HID), jnp.float32),
            pltpu.SemaphoreType.DMA,
            pltpu.SemaphoreType.DMA,
            pltpu.SemaphoreType.DMA,
            pltpu.SemaphoreType.DMA,
        ],
    )
    def msg_sc(x_hbm, w_hbm, src_hbm, dst_hbm, zero_hbm, out_hbm,
               agg_sp, srcv, dstv, xv0, xv1, wv0, wv1, sg0, sg1, ss0, ss1):
        c = lax.axis_index("c")
        s = lax.axis_index("s")
        wid = c * NSUB + s
        xv = (xv0, xv1)
        wv = (wv0, wv1)
        sg = (sg0, sg1)
        ss = (ss0, ss1)

        row0 = s * rps
        rsl = pl.ds(row0, rps)
        pltpu.sync_copy(zero_hbm.at[rsl], agg_sp.at[rsl])
        plsc.subcore_barrier()

        base0 = wid * cpt * CHUNK
        cb0 = wid * cpt

        def load_batch(pr, pp):
            pltpu.sync_copy(src_hbm.at[pl.ds(cb0 + 2 * pr, 2)], srcv.at[pp])
            pltpu.sync_copy(dst_hbm.at[pl.ds(cb0 + 2 * pr, 2)], dstv.at[pp])

        def chunk_step(k, b, bcur, bnxt, do_swait, do_issue):
            eb = base0 + k * CHUNK
            pltpu.make_async_copy(x_hbm.at[srcv.at[0, b]], xv[b],
                                  sg[b]).wait()
            if do_swait:
                pltpu.make_async_copy(wv[b], agg_sp.at[dstv.at[0, b]],
                                      ss[b]).wait()
            pltpu.sync_copy(w_hbm.at[pl.ds(eb, CHUNK)], wv[b])

            def mbody(e, cc):
                for j in range(HID // 16):
                    sl = pl.ds(j * 16, 16)
                    wv[b][e, sl] = wv[b][e, sl] * xv[b][e, sl]
                return cc

            lax.fori_loop(0, CHUNK, mbody, 0)
            pltpu.async_copy(wv[b], agg_sp.at[dstv.at[bcur, b]], ss[b],
                             add=True)
            if do_issue:
                pltpu.async_copy(x_hbm.at[srcv.at[bnxt, b]], xv[b], sg[b])

        npr = cpt // 2
        load_batch(0, 0)
        for b in range(2):
            pltpu.async_copy(x_hbm.at[srcv.at[0, b]], xv[b], sg[b])
        if npr > 1:
            load_batch(1, 1)
        for k in range(min(2, cpt)):
            chunk_step(k, k % 2, 0, 1, False, k < cpt - 2)
        npairs = max(0, npr - 2)

        def pbody(g, carry):
            pr = g + 1
            pp = pr % 2
            load_batch(pr + 1, 1 - pp)
            chunk_step(2 * pr, 0, pp, 1 - pp, True, True)
            chunk_step(2 * pr + 1, 1, pp, 1 - pp, True, True)
            return carry

        lax.fori_loop(0, npairs, pbody, 0)
        if npr > 1:
            pr = npr - 1
            pp = pr % 2
            for k in range(2 * pr, cpt):
                chunk_step(k, k % 2, pp, 1 - pp, True, k < cpt - 2)
        for k in range(max(0, cpt - 2), cpt):
            b = k % 2
            pltpu.make_async_copy(wv[b], agg_sp.at[dstv.at[0, b]],
                                  ss[b]).wait()
        plsc.subcore_barrier()

        pltpu.sync_copy(agg_sp.at[rsl], out_hbm.at[c, rsl])

    return msg_sc


def _update_body(p_ref, h_ref, l2_ref, l2b_ref, ow_ref, ob_ref, l1n_ref,
                 hout_ref, xout_ref):
    agg = p_ref[0] + p_ref[1]
    y = jnp.dot(agg, l2_ref[...], preferred_element_type=jnp.float32)
    y = _ssp(y + l2b_ref[...])
    y = jnp.dot(y, ow_ref[...], preferred_element_type=jnp.float32) + ob_ref[...]
    h = h_ref[...] + y
    hout_ref[...] = h
    xout_ref[...] = jnp.dot(h, l1n_ref[...], preferred_element_type=jnp.float32)


def _update(part, h, l2, l2b, ow, ob, l1n, bn):
    n = h.shape[0]
    return pl.pallas_call(
        _update_body,
        grid=(n // bn,),
        in_specs=[
            pl.BlockSpec((NCORE, bn, HID), lambda i: (0, i, 0)),
            pl.BlockSpec((bn, HID), lambda i: (i, 0)),
            pl.BlockSpec((HID, HID), lambda i: (0, 0)),
            pl.BlockSpec((1, HID), lambda i: (0, 0)),
            pl.BlockSpec((HID, HID), lambda i: (0, 0)),
            pl.BlockSpec((1, HID), lambda i: (0, 0)),
            pl.BlockSpec((HID, HID), lambda i: (0, 0)),
        ],
        out_specs=(
            pl.BlockSpec((bn, HID), lambda i: (i, 0)),
            pl.BlockSpec((bn, HID), lambda i: (i, 0)),
        ),
        out_shape=(
            jax.ShapeDtypeStruct((n, HID), jnp.float32),
            jax.ShapeDtypeStruct((n, HID), jnp.float32),
        ),
    )(part, h, l2, l2b, ow, ob, l1n)


def kernel(z, edge_index, edge_vec, emb, mlp_W1, mlp_b1, mlp_W2, mlp_b2,
           lin1_W, lin2_W, lin2_b, out_W, out_b):
    n = z.shape[0]
    e = edge_index.shape[1]
    nl = mlp_W1.shape[0]

    z = z.astype(jnp.int32)
    edge_index = edge_index.astype(jnp.int32)
    edge_vec = edge_vec.astype(jnp.float32)

    egrain = (2 * NTILE * CHUNK) * BE // math.gcd(2 * NTILE * CHUNK, BE)
    epad = -(-e // egrain) * egrain
    cpt = epad // (NTILE * CHUNK)
    pad = epad - e
    src = jnp.pad(edge_index[0], (0, pad))
    dst = jnp.pad(edge_index[1], (0, pad))
    evp = jnp.pad(edge_vec, ((0, pad), (0, 0)), constant_values=10.0 * RCUT)

    w_layers = _compute_filters(evp, mlp_W1, mlp_b1, mlp_W2, mlp_b2)

    npad = -(-n // (NSUB * 8)) * (NSUB * 8)
    bn = npad // 4 if (npad // 4) % 8 == 0 else npad // NSUB
    zp = jnp.pad(z, (0, npad - n), constant_values=emb.shape[0] + 7)
    h, x = _embed(zp.reshape(npad, 1), emb, lin1_W[0], bn)

    msg_sc = _make_msg_sc(npad, epad, cpt)
    zero_tab = jnp.zeros((npad, HID), jnp.float32)
    src2 = src.reshape(epad // CHUNK, CHUNK)
    dst2 = dst.reshape(epad // CHUNK, CHUNK)
    for i in range(nl):
        part = msg_sc(x, w_layers[i], src2, dst2, zero_tab)
        l1n = lin1_W[(i + 1) % nl]
        h, x = _update(part, h, lin2_W[i], lin2_b[i].reshape(1, HID),
                       out_W[i], out_b[i].reshape(1, HID), l1n, bn)
    return h[:n]


# speedup vs baseline: 1.1037x; 1.0452x over previous
import functools
import math

import jax
import jax.numpy as jnp
from jax import lax
from jax.experimental import pallas as pl
from jax.experimental.pallas import tpu as pltpu
from jax.experimental.pallas import tpu_sc as plsc

HID = 128
NRBF = 50
RCUT = 5.0
LN2 = 0.6931471805599453

NCORE = 2
NSUB = 16
NTILE = NCORE * NSUB
CHUNK = 96

BE = 2048


def _ssp(x):
    m = jnp.maximum(x, 0.0)
    return m + jnp.log(jnp.exp(x - m) + jnp.exp(-m)) - LN2


def _filters_body(nl, ev_ref, w1_ref, b1_ref, w2_ref, b2_ref, *out_refs):
    ev = ev_ref[...]
    d = jnp.sqrt(jnp.sum(ev * ev, axis=1, keepdims=True))
    delta = RCUT / (NRBF - 1)
    offs = lax.broadcasted_iota(jnp.int32, (1, NRBF), 1).astype(jnp.float32) * delta
    coeff = -0.5 / (delta * delta)
    rbf = jnp.exp(coeff * (d - offs) ** 2)
    cutoff = 0.5 * (jnp.cos(d * (jnp.pi / RCUT)) + 1.0)
    cutoff = cutoff * (d < RCUT).astype(jnp.float32)
    for i in range(nl):
        t = jnp.dot(rbf, w1_ref[i], preferred_element_type=jnp.float32)
        t = _ssp(t + b1_ref[i])
        w = jnp.dot(t, w2_ref[i], preferred_element_type=jnp.float32)
        w = (w + b2_ref[i]) * cutoff
        out_refs[i][...] = w


def _compute_filters(evp, w1, b1, w2, b2):
    nl = w1.shape[0]
    epad = evp.shape[0]
    grid = (epad // BE,)
    out_shape = tuple(
        jax.ShapeDtypeStruct((epad, HID), jnp.float32) for _ in range(nl))
    return pl.pallas_call(
        functools.partial(_filters_body, nl),
        grid=grid,
        in_specs=[
            pl.BlockSpec((BE, 3), lambda i: (i, 0)),
            pl.BlockSpec(w1.shape, lambda i: (0, 0, 0)),
            pl.BlockSpec(b1.shape, lambda i: (0, 0)),
            pl.BlockSpec(w2.shape, lambda i: (0, 0, 0)),
            pl.BlockSpec(b2.shape, lambda i: (0, 0)),
        ],
        out_specs=tuple(
            pl.BlockSpec((BE, HID), lambda i: (i, 0)) for _ in range(nl)),
        out_shape=out_shape,
    )(evp, w1, b1, w2, b2)


def _embed_body(z_ref, emb_ref, l1_ref, h_ref, x_ref):
    zb = z_ref[...]
    nv = emb_ref.shape[0]
    oh = (zb == lax.broadcasted_iota(jnp.int32, (1, nv), 1)).astype(jnp.float32)
    h = jnp.dot(oh, emb_ref[...], preferred_element_type=jnp.float32)
    h_ref[...] = h
    x_ref[...] = jnp.dot(h, l1_ref[...], preferred_element_type=jnp.float32)


def _embed(z2, emb, l1, bn):
    n = z2.shape[0]
    return pl.pallas_call(
        _embed_body,
        grid=(n // bn,),
        in_specs=[
            pl.BlockSpec((bn, 1), lambda i: (i, 0)),
            pl.BlockSpec(emb.shape, lambda i: (0, 0)),
            pl.BlockSpec(l1.shape, lambda i: (0, 0)),
        ],
        out_specs=(
            pl.BlockSpec((bn, HID), lambda i: (i, 0)),
            pl.BlockSpec((bn, HID), lambda i: (i, 0)),
        ),
        out_shape=(
            jax.ShapeDtypeStruct((n, HID), jnp.float32),
            jax.ShapeDtypeStruct((n, HID), jnp.float32),
        ),
    )(z2, emb, l1)


def _make_msg_sc(n, epad, cpt):
    rps = n // NSUB
    mesh = plsc.VectorSubcoreMesh(core_axis_name="c", subcore_axis_name="s",
                                  num_cores=NCORE, num_subcores=NSUB)

    @functools.partial(
        pl.kernel,
        out_type=jax.ShapeDtypeStruct((NCORE, n, HID), jnp.float32),
        mesh=mesh,
        scratch_types=[
            pltpu.VMEM_SHARED((n, HID), jnp.float32),
            pltpu.VMEM((2, 2, CHUNK), jnp.int32),
            pltpu.VMEM((2, 2, CHUNK), jnp.int32),
            pltpu.VMEM((CHUNK, HID), jnp.float32),
            pltpu.VMEM((CHUNK, HID), jnp.float32),
            pltpu.VMEM((CHUNK, HID), jnp.float32),
            pltpu.VMEM((CHUNK, HID), jnp.float32),
            pltpu.SemaphoreType.DMA,
            pltpu.SemaphoreType.DMA,
            pltpu.SemaphoreType.DMA,
            pltpu.SemaphoreType.DMA,
            pltpu.SemaphoreType.DMA,
            pltpu.SemaphoreType.DMA,
        ],
    )
    def msg_sc(x_hbm, w_hbm, src_hbm, dst_hbm, zero_hbm, out_hbm,
               agg_sp, srcv, dstv, xv0, xv1, wv0, wv1,
               sg0, sg1, ss0, ss1, sw0, sw1):
        c = lax.axis_index("c")
        s = lax.axis_index("s")
        wid = c * NSUB + s
        xv = (xv0, xv1)
        wv = (wv0, wv1)
        sg = (sg0, sg1)
        ss = (ss0, ss1)
        sw = (sw0, sw1)

        row0 = s * rps
        rsl = pl.ds(row0, rps)
        pltpu.sync_copy(zero_hbm.at[rsl], agg_sp.at[rsl])
        plsc.subcore_barrier()

        base0 = wid * cpt * CHUNK
        cb0 = wid * cpt

        def load_batch(pr, pp):
            pltpu.sync_copy(src_hbm.at[pl.ds(cb0 + 2 * pr, 2)], srcv.at[pp])
            pltpu.sync_copy(dst_hbm.at[pl.ds(cb0 + 2 * pr, 2)], dstv.at[pp])

        def issue_w(k, b):
            pltpu.async_copy(w_hbm.at[pl.ds(base0 + k * CHUNK, CHUNK)],
                             wv[b], sw[b])

        def chunk_step(k, b, bcur, bnxt, do_swait, do_issue, do_w):
            pltpu.make_async_copy(x_hbm.at[srcv.at[0, b]], xv[b],
                                  sg[b]).wait()
            pltpu.make_async_copy(w_hbm.at[pl.ds(0, CHUNK)], wv[b],
                                  sw[b]).wait()

            def mbody(e, cc):
                for j in range(HID // 16):
                    sl = pl.ds(j * 16, 16)
                    wv[b][e, sl] = wv[b][e, sl] * xv[b][e, sl]
                return cc

            lax.fori_loop(0, CHUNK, mbody, 0)
            pltpu.async_copy(wv[b], agg_sp.at[dstv.at[bcur, b]], ss[b],
                             add=True)
            if do_issue:
                pltpu.async_copy(x_hbm.at[srcv.at[bnxt, b]], xv[b], sg[b])
            if do_swait:
                pltpu.make_async_copy(wv[1 - b], agg_sp.at[dstv.at[0, 1 - b]],
                                      ss[1 - b]).wait()
            if do_w:
                issue_w(k + 1, 1 - b)

        npr = cpt // 2
        load_batch(0, 0)
        for b in range(2):
            pltpu.async_copy(x_hbm.at[srcv.at[0, b]], xv[b], sg[b])
        issue_w(0, 0)
        if npr > 1:
            load_batch(1, 1)
        chunk_step(0, 0, 0, 1, False, cpt > 2, cpt > 1)
        if cpt > 1:
            chunk_step(1, 1, 0, 1, True, cpt > 3, cpt > 2)
        npairs = max(0, npr - 2)

        def pbody(g, carry):
            pr = g + 1
            pp = pr % 2
            load_batch(pr + 1, 1 - pp)
            chunk_step(2 * pr, 0, pp, 1 - pp, True, True, True)
            chunk_step(2 * pr + 1, 1, pp, 1 - pp, True, True, True)
            return carry

        lax.fori_loop(0, npairs, pbody, 0)
        if npr > 1:
            pr = npr - 1
            pp = pr % 2
            for k in range(2 * pr, cpt):
                chunk_step(k, k % 2, pp, 1 - pp, True, k < cpt - 2,
                           k + 1 < cpt)
        b = (cpt - 1) % 2
        pltpu.make_async_copy(wv[b], agg_sp.at[dstv.at[0, b]], ss[b]).wait()
        plsc.subcore_barrier()

        pltpu.sync_copy(agg_sp.at[rsl], out_hbm.at[c, rsl])

    return msg_sc


def _update_body(p_ref, h_ref, l2_ref, l2b_ref, ow_ref, ob_ref, l1n_ref,
                 hout_ref, xout_ref):
    agg = p_ref[0] + p_ref[1]
    y = jnp.dot(agg, l2_ref[...], preferred_element_type=jnp.float32)
    y = _ssp(y + l2b_ref[...])
    y = jnp.dot(y, ow_ref[...], preferred_element_type=jnp.float32) + ob_ref[...]
    h = h_ref[...] + y
    hout_ref[...] = h
    xout_ref[...] = jnp.dot(h, l1n_ref[...], preferred_element_type=jnp.float32)


def _update(part, h, l2, l2b, ow, ob, l1n, bn):
    n = h.shape[0]
    return pl.pallas_call(
        _update_body,
        grid=(n // bn,),
        in_specs=[
            pl.BlockSpec((NCORE, bn, HID), lambda i: (0, i, 0)),
            pl.BlockSpec((bn, HID), lambda i: (i, 0)),
            pl.BlockSpec((HID, HID), lambda i: (0, 0)),
            pl.BlockSpec((1, HID), lambda i: (0, 0)),
            pl.BlockSpec((HID, HID), lambda i: (0, 0)),
            pl.BlockSpec((1, HID), lambda i: (0, 0)),
            pl.BlockSpec((HID, HID), lambda i: (0, 0)),
        ],
        out_specs=(
            pl.BlockSpec((bn, HID), lambda i: (i, 0)),
            pl.BlockSpec((bn, HID), lambda i: (i, 0)),
        ),
        out_shape=(
            jax.ShapeDtypeStruct((n, HID), jnp.float32),
            jax.ShapeDtypeStruct((n, HID), jnp.float32),
        ),
    )(part, h, l2, l2b, ow, ob, l1n)


def kernel(z, edge_index, edge_vec, emb, mlp_W1, mlp_b1, mlp_W2, mlp_b2,
           lin1_W, lin2_W, lin2_b, out_W, out_b):
    n = z.shape[0]
    e = edge_index.shape[1]
    nl = mlp_W1.shape[0]

    z = z.astype(jnp.int32)
    edge_index = edge_index.astype(jnp.int32)
    edge_vec = edge_vec.astype(jnp.float32)

    egrain = (2 * NTILE * CHUNK) * BE // math.gcd(2 * NTILE * CHUNK, BE)
    epad = -(-e // egrain) * egrain
    cpt = epad // (NTILE * CHUNK)
    pad = epad - e
    src = jnp.pad(edge_index[0], (0, pad))
    dst = jnp.pad(edge_index[1], (0, pad))
    evp = jnp.pad(edge_vec, ((0, pad), (0, 0)), constant_values=10.0 * RCUT)

    w_layers = _compute_filters(evp, mlp_W1, mlp_b1, mlp_W2, mlp_b2)

    npad = -(-n // (NSUB * 8)) * (NSUB * 8)
    bn = npad // 4 if (npad // 4) % 8 == 0 else npad // NSUB
    zp = jnp.pad(z, (0, npad - n), constant_values=emb.shape[0] + 7)
    h, x = _embed(zp.reshape(npad, 1), emb, lin1_W[0], bn)

    msg_sc = _make_msg_sc(npad, epad, cpt)
    zero_tab = jnp.zeros((npad, HID), jnp.float32)
    src2 = src.reshape(epad // CHUNK, CHUNK)
    dst2 = dst.reshape(epad // CHUNK, CHUNK)
    for i in range(nl):
        part = msg_sc(x, w_layers[i], src2, dst2, zero_tab)
        l1n = lin1_W[(i + 1) % nl]
        h, x = _update(part, h, lin2_W[i], lin2_b[i].reshape(1, HID),
                       out_W[i], out_b[i].reshape(1, HID), l1n, bn)
    return h[:n]
